# baseline (speedup 1.0000x reference)
import functools

import jax
import jax.numpy as jnp
from jax import lax
from jax.experimental import pallas as pl
from jax.experimental.pallas import tpu as pltpu
from jax.experimental.pallas import tpu_sc as plsc

_B, _H, _W, _C = 32, 16, 16, 64
_N = _H * _W
_R = _B * _N
_E = _N * (_N - 1)
_VREGS = _E // 16
_UNROLL = 16


def _template_kernel(tmpl_ref):
    k = jax.lax.broadcasted_iota(jnp.int32, (1, _E), 1)
    i = jnp.right_shift(k + jnp.right_shift(k, 8) + 1, 8)
    j = k - ((i << 8) - i)
    tmpl_ref[0:1] = i
    tmpl_ref[1:2] = j + (j >= i).astype(jnp.int32)


def _nodes_kernel(x_ref, nodes_ref, batch_ref):
    b = pl.program_id(0)
    rows = jax.lax.broadcasted_iota(jnp.int32, (_N, 1), 0)
    hr = jnp.right_shift(rows, 4).astype(jnp.float32) * (1.0 / (_H - 1))
    wc = jnp.bitwise_and(rows, _W - 1).astype(jnp.float32) * (1.0 / (_W - 1))
    nodes_ref[...] = jnp.concatenate([x_ref[...], hr, wc], axis=1)
    batch_ref[...] = jnp.full((_N, 1), b, dtype=jnp.int32)


def _sc_edges_kernel(tmpl_hbm, out_hbm, buf):
    b = lax.axis_index("c") * 16 + lax.axis_index("s")
    offv = lax.broadcast(b * _N, (16,))

    def body(it, carry):
        base = it * (16 * _UNROLL)
        for v in range(_UNROLL):
            sl = pl.ds(base + v * 16, 16)
            buf[sl] = buf[sl] + offv
        return carry

    for r in range(2):
        pltpu.sync_copy(tmpl_hbm.at[pl.ds(r * _E, _E)], buf)
        lax.fori_loop(0, _VREGS // _UNROLL, body, 0)
        pltpu.sync_copy(buf, out_hbm.at[pl.ds((r * _B + b) * _E, _E)])


def kernel(x, y):
    x2d = x.reshape(_R, _C)
    tmpl = pl.pallas_call(
        _template_kernel,
        out_shape=jax.ShapeDtypeStruct((2, _E), jnp.int32),
    )()

    sc_edges = functools.partial(
        pl.kernel,
        mesh=plsc.VectorSubcoreMesh(core_axis_name="c", subcore_axis_name="s"),
        out_type=jax.ShapeDtypeStruct((2 * _B * _E,), jnp.int32),
        scratch_types=[pltpu.VMEM((_E,), jnp.int32)],
    )(_sc_edges_kernel)
    edge_index = sc_edges(tmpl.reshape(2 * _E)).reshape(2, _B * _E)

    nodes, batch2 = pl.pallas_call(
        _nodes_kernel,
        grid=(_B,),
        in_specs=[pl.BlockSpec((_N, _C), lambda b: (b, 0))],
        out_specs=[
            pl.BlockSpec((_N, _C + 2), lambda b: (b, 0)),
            pl.BlockSpec((_N, 1), lambda b: (b, 0)),
        ],
        out_shape=[
            jax.ShapeDtypeStruct((_R, _C + 2), jnp.float32),
            jax.ShapeDtypeStruct((_R, 1), jnp.int32),
        ],
    )(x2d)
    batch_vec = batch2.reshape(_R)
    y_out = y.reshape(_B, -1)
    return nodes, edge_index, batch_vec, y_out


# speedup vs baseline: 3.3629x; 3.3629x over previous
import jax
import jax.numpy as jnp
from jax.experimental import pallas as pl

_B, _H, _W, _C = 32, 16, 16, 64
_N = _H * _W
_R = _B * _N
_E = _N * (_N - 1)
_G = 4
_STEPS = _B // _G


def _template_kernel(tmpl_ref):
    k = jax.lax.broadcasted_iota(jnp.int32, (1, _E), 1)
    i = jnp.right_shift(k + jnp.right_shift(k, 8) + 1, 8)
    j = k - ((i << 8) - i)
    src = i
    dst = j + (j >= i).astype(jnp.int32)
    for q in range(_G):
        tmpl_ref[0:1, q * _E:(q + 1) * _E] = src + q * _N
        tmpl_ref[1:2, q * _E:(q + 1) * _E] = dst + q * _N
    del k


def _build_kernel(tmpl_ref, x_ref, edges_ref, nodes_ref, batch_ref):
    g = pl.program_id(0)
    edges_ref[...] = tmpl_ref[...] + g * (_G * _N)

    rows = jax.lax.broadcasted_iota(jnp.int32, (_G * _N, 1), 0)
    p = jnp.bitwise_and(rows, _N - 1)
    hr = jnp.right_shift(p, 4).astype(jnp.float32) * (1.0 / (_H - 1))
    wc = jnp.bitwise_and(p, _W - 1).astype(jnp.float32) * (1.0 / (_W - 1))
    nodes_ref[...] = jnp.concatenate([x_ref[...], hr, wc], axis=1)
    batch_ref[...] = g * _G + jnp.right_shift(rows, 8)


def kernel(x, y):
    x2d = x.reshape(_R, _C)
    tmpl = pl.pallas_call(
        _template_kernel,
        out_shape=jax.ShapeDtypeStruct((2, _G * _E), jnp.int32),
    )()
    edge_index, nodes, batch2 = pl.pallas_call(
        _build_kernel,
        grid=(_STEPS,),
        in_specs=[
            pl.BlockSpec((2, _G * _E), lambda g: (0, 0)),
            pl.BlockSpec((_G * _N, _C), lambda g: (g, 0)),
        ],
        out_specs=[
            pl.BlockSpec((2, _G * _E), lambda g: (0, g)),
            pl.BlockSpec((_G * _N, _C + 2), lambda g: (g, 0)),
            pl.BlockSpec((_G * _N, 1), lambda g: (g, 0)),
        ],
        out_shape=[
            jax.ShapeDtypeStruct((2, _B * _E), jnp.int32),
            jax.ShapeDtypeStruct((_R, _C + 2), jnp.float32),
            jax.ShapeDtypeStruct((_R, 1), jnp.int32),
        ],
    )(tmpl, x2d)
    batch_vec = batch2.reshape(_R)
    y_out = y.reshape(_B, -1)
    return nodes, edge_index, batch_vec, y_out


# speedup vs baseline: 3.6747x; 1.0927x over previous
import jax
import jax.numpy as jnp
from jax.experimental import pallas as pl
from jax.experimental.pallas import tpu as pltpu

_B, _H, _W, _C = 32, 16, 16, 64
_N = _H * _W
_R = _B * _N
_E = _N * (_N - 1)
_G = 4
_STEPS = _B // _G


def _build_kernel(x_ref, edges_ref, nodes_ref, batch_ref, tmpl_ref):
    g = pl.program_id(0)

    @pl.when(g == 0)
    def _init_template():
        k = jax.lax.broadcasted_iota(jnp.int32, (1, _E), 1)
        i = jnp.right_shift(k + jnp.right_shift(k, 8) + 1, 8)
        j = k - ((i << 8) - i)
        src = i
        dst = j + (j >= i).astype(jnp.int32)
        for q in range(_G):
            tmpl_ref[0:1, q * _E:(q + 1) * _E] = src + q * _N
            tmpl_ref[1:2, q * _E:(q + 1) * _E] = dst + q * _N

    edges_ref[...] = tmpl_ref[...] + g * (_G * _N)

    rows = jax.lax.broadcasted_iota(jnp.int32, (_G * _N, 1), 0)
    p = jnp.bitwise_and(rows, _N - 1)
    hr = jnp.right_shift(p, 4).astype(jnp.float32) * (1.0 / (_H - 1))
    wc = jnp.bitwise_and(p, _W - 1).astype(jnp.float32) * (1.0 / (_W - 1))
    nodes_ref[...] = jnp.concatenate([x_ref[...], hr, wc], axis=1)
    batch_ref[...] = g * _G + jnp.right_shift(rows, 8)


def kernel(x, y):
    x2d = x.reshape(_R, _C)
    edge_index, nodes, batch2 = pl.pallas_call(
        _build_kernel,
        grid=(_STEPS,),
        in_specs=[pl.BlockSpec((_G * _N, _C), lambda g: (g, 0))],
        out_specs=[
            pl.BlockSpec((2, _G * _E), lambda g: (0, g)),
            pl.BlockSpec((_G * _N, _C + 2), lambda g: (g, 0)),
            pl.BlockSpec((_G * _N, 1), lambda g: (g, 0)),
        ],
        out_shape=[
            jax.ShapeDtypeStruct((2, _B * _E), jnp.int32),
            jax.ShapeDtypeStruct((_R, _C + 2), jnp.float32),
            jax.ShapeDtypeStruct((_R, 1), jnp.int32),
        ],
        scratch_shapes=[pltpu.VMEM((2, _G * _E), jnp.int32)],
    )(x2d)
    batch_vec = batch2.reshape(_R)
    y_out = y.reshape(_B, -1)
    return nodes, edge_index, batch_vec, y_out
